# Initial kernel scaffold; baseline (speedup 1.0000x reference)
import jax
import jax.numpy as jnp
from jax.experimental import pallas as pl


def kernel(token_ids, weight):
    raise NotImplementedError("write your pallas kernel here")


# speedup vs baseline: 4.8060x; 4.8060x over previous
import functools

import jax
import jax.numpy as jnp
from jax import lax
from jax.experimental import pallas as pl
from jax.experimental.pallas import tpu as pltpu
from jax.experimental.pallas import tpu_sc as plsc

_NUM_CORES = 2
_NUM_SUBCORES = 16
_NW = _NUM_CORES * _NUM_SUBCORES

_CHUNK = 1024


@functools.cache
def _make_lookup(B, D, chunk):
    bpw = B // _NW
    nchunk = bpw // chunk
    mesh = plsc.VectorSubcoreMesh(core_axis_name="c", subcore_axis_name="s")

    @functools.partial(
        pl.kernel,
        out_type=jax.ShapeDtypeStruct((B, D), jnp.float32),
        mesh=mesh,
        scratch_types=[
            pltpu.VMEM((chunk,), jnp.int32),
            pltpu.VMEM((chunk, D), jnp.float32),
            pltpu.SemaphoreType.DMA,
        ],
        compiler_params=pltpu.CompilerParams(use_tc_tiling_on_sc=False),
    )
    def k(idx_hbm, table_hbm, out_hbm, idx_v, rows_v, sem):
        wid = lax.axis_index("s") * _NUM_CORES + lax.axis_index("c")
        base = wid * bpw

        def body(i, carry):
            off = base + i * chunk
            pltpu.sync_copy(idx_hbm.at[pl.ds(off, chunk)], idx_v)
            pltpu.async_copy(table_hbm.at[idx_v], rows_v, sem).wait()
            pltpu.sync_copy(rows_v, out_hbm.at[pl.ds(off, chunk)])
            return carry

        lax.fori_loop(0, nchunk, body, 0)

    return k


def kernel(token_ids, weight):
    S, T = token_ids.shape
    _, D = weight.shape
    B = S * T
    flat = token_ids.reshape(B).astype(jnp.int32)
    out = _make_lookup(B, D, _CHUNK)(flat, weight)
    return out.reshape(S, T, D)


# speedup vs baseline: 5.0483x; 1.0504x over previous
import functools

import jax
import jax.numpy as jnp
from jax import lax
from jax.experimental import pallas as pl
from jax.experimental.pallas import tpu as pltpu
from jax.experimental.pallas import tpu_sc as plsc

_NUM_CORES = 2
_NUM_SUBCORES = 16
_NW = _NUM_CORES * _NUM_SUBCORES

_CHUNK = 800
_NBUF = 4


@functools.cache
def _make_lookup(B, D, chunk, nbuf):
    bpw = B // _NW
    nchunk = bpw // chunk
    ngroup = nchunk // nbuf
    assert bpw % chunk == 0 and nchunk % nbuf == 0
    mesh = plsc.VectorSubcoreMesh(core_axis_name="c", subcore_axis_name="s")

    @functools.partial(
        pl.kernel,
        out_type=jax.ShapeDtypeStruct((B, D), jnp.float32),
        mesh=mesh,
        scratch_types=[
            pltpu.VMEM((nbuf, chunk), jnp.int32),
            pltpu.VMEM((nbuf, chunk, D), jnp.float32),
            pltpu.SemaphoreType.DMA,
            pltpu.SemaphoreType.DMA,
            pltpu.SemaphoreType.DMA,
        ],
        compiler_params=pltpu.CompilerParams(use_tc_tiling_on_sc=False),
    )
    def k(idx_hbm, table_hbm, out_hbm, idx_v, rows_v, sem_i, sem_g, sem_w):
        wid = lax.axis_index("s") * _NUM_CORES + lax.axis_index("c")
        base = wid * bpw

        def idx_src(ci):
            return idx_hbm.at[pl.ds(base + ci * chunk, chunk)]

        def out_dst(ci):
            return out_hbm.at[pl.ds(base + ci * chunk, chunk)]

        for b in range(nbuf):
            pltpu.async_copy(idx_src(b), idx_v.at[b], sem_i)

        def group(g, carry):
            for b in range(nbuf):
                @pl.when(g > 0)
                def _():
                    pltpu.make_async_copy(rows_v.at[b], out_dst(0), sem_w).wait()

                pltpu.make_async_copy(idx_src(0), idx_v.at[b], sem_i).wait()
                pltpu.async_copy(table_hbm.at[idx_v.at[b]], rows_v.at[b], sem_g)
            for b in range(nbuf):
                ci = g * nbuf + b
                pltpu.make_async_copy(
                    table_hbm.at[idx_v.at[b]], rows_v.at[b], sem_g
                ).wait()
                pltpu.async_copy(rows_v.at[b], out_dst(ci), sem_w)

                @pl.when(g + 1 < ngroup)
                def _():
                    pltpu.async_copy(idx_src(ci + nbuf), idx_v.at[b], sem_i)

            return carry

        lax.fori_loop(0, ngroup, group, 0)
        for b in range(nbuf):
            pltpu.make_async_copy(rows_v.at[b], out_dst(0), sem_w).wait()

    return k


def kernel(token_ids, weight):
    S, T = token_ids.shape
    _, D = weight.shape
    B = S * T
    flat = token_ids.reshape(B).astype(jnp.int32)
    out = _make_lookup(B, D, _CHUNK, _NBUF)(flat, weight)
    return out.reshape(S, T, D)
